# baseline (speedup 1.0000x reference)
import functools
import math

import jax
import jax.numpy as jnp
from jax.experimental import pallas as pl
from jax.experimental.pallas import tpu as pltpu
from jax.experimental.pallas import tpu_sc as plsc

S = 4096
DIM = 2048
NH, HD = 16, 128
HH = HD // 2
BS = 8
TB = S // BS
SINK_B = 4
WIN_B = 4
CUR_BLOCK = TB - 1
WIN_START = CUR_BLOCK - WIN_B + 1
MB = 512 // BS
KL = SINK_B + WIN_B + MB
KSEL = KL * BS
SCALE = 1.0 / math.sqrt(HD)
NEG = -1e30
KOFF = NH * HD
VOFF = 2 * NH * HD


def _pair_swap(z):
    ev = jax.lax.broadcasted_iota(jnp.int32, z.shape, 1) % 2 == 0
    return jnp.where(ev, jnp.roll(z, -1, axis=1), jnp.roll(z, 1, axis=1))


def _qkvrope_kernel(x_ref, w_ref, ca_ref, sb_ref,
                    qr_ref, kr_ref, v_ref, kb_ref, ql_ref):
    i = pl.program_id(0)
    acc = jax.lax.dot_general(
        x_ref[:, :].astype(jnp.bfloat16), w_ref[:, :],
        (((1,), (0,)), ((), ())),
        preferred_element_type=jnp.float32)
    bm = acc.shape[0]
    ca = jnp.concatenate([ca_ref[:, :]] * NH, axis=1)
    sb = jnp.concatenate([sb_ref[:, :]] * NH, axis=1)
    q = acc[:, :KOFF]
    k = acc[:, KOFF:VOFF]
    qrot = q * ca + _pair_swap(q) * sb
    krot = k * ca + _pair_swap(k) * sb
    qr_ref[:, :] = qrot.astype(jnp.bfloat16)
    kr_ref[:, :] = krot.astype(jnp.bfloat16)
    kb_ref[:, :] = jnp.mean(krot.reshape(bm // BS, BS, KOFF), axis=1)
    v_ref[:, :] = acc[:, VOFF:].astype(jnp.bfloat16)

    @pl.when(i == pl.num_programs(0) - 1)
    def _():
        ql_ref[:, :] = acc[bm - 8:, :KOFF]


def _qkv_rope(x2, w_all, ca2, sb2, bm):
    return pl.pallas_call(
        _qkvrope_kernel,
        grid=(S // bm,),
        in_specs=[pl.BlockSpec((bm, DIM), lambda i: (i, 0)),
                  pl.BlockSpec((DIM, 3 * NH * HD), lambda i: (0, 0)),
                  pl.BlockSpec((bm, HD), lambda i: (i, 0)),
                  pl.BlockSpec((bm, HD), lambda i: (i, 0))],
        out_specs=[pl.BlockSpec((bm, NH * HD), lambda i: (i, 0)),
                   pl.BlockSpec((bm, NH * HD), lambda i: (i, 0)),
                   pl.BlockSpec((bm, NH * HD), lambda i: (i, 0)),
                   pl.BlockSpec((bm // BS, NH * HD), lambda i: (i, 0)),
                   pl.BlockSpec((8, NH * HD), lambda i: (0, 0))],
        out_shape=[jax.ShapeDtypeStruct((S, NH * HD), jnp.bfloat16),
                   jax.ShapeDtypeStruct((S, NH * HD), jnp.bfloat16),
                   jax.ShapeDtypeStruct((S, NH * HD), jnp.bfloat16),
                   jax.ShapeDtypeStruct((TB, NH * HD), jnp.float32),
                   jax.ShapeDtypeStruct((8, NH * HD), jnp.float32)],
    )(x2, w_all, ca2, sb2)


def _route_kernel(kb_ref, ql_ref, ca_ref, sb_ref, top_ref):
    ca = ca_ref[:, :]
    sb = sb_ref[:, :]
    qraw = ql_ref[7:8, :]
    cols = []
    for h in range(NH):
        qh = qraw[:, h * HD:(h + 1) * HD]
        qlh = qh * ca + _pair_swap(qh) * sb
        kb16 = kb_ref[:, h * HD:(h + 1) * HD].astype(jnp.bfloat16)
        ql16 = qlh.astype(jnp.bfloat16).astype(jnp.float32)
        cols.append(jnp.sum(kb16.astype(jnp.float32) * ql16,
                            axis=1, keepdims=True))
    scores = jnp.concatenate(cols, axis=1)
    rid = jax.lax.broadcasted_iota(jnp.int32, (TB, NH), 0)
    excl = (rid < SINK_B) | (rid >= WIN_START)
    scores = jnp.where(excl, NEG, scores)

    def body(j, sc):
        idx = jnp.argmax(sc, axis=0).astype(jnp.int32)
        top_ref[pl.ds(j, 1), :] = idx[None, :]
        hit = rid == idx[None, :]
        return jnp.where(hit, -jnp.inf, sc)

    jax.lax.fori_loop(0, MB, body, scores)


def _route(kb2, ql8, ca_last, sb_last):
    return pl.pallas_call(
        _route_kernel,
        grid=(1,),
        in_specs=[pl.BlockSpec((TB, NH * HD), lambda i: (0, 0)),
                  pl.BlockSpec((8, NH * HD), lambda i: (0, 0)),
                  pl.BlockSpec((1, HD), lambda i: (0, 0)),
                  pl.BlockSpec((1, HD), lambda i: (0, 0))],
        out_specs=pl.BlockSpec((MB, NH), lambda i: (0, 0)),
        out_shape=jax.ShapeDtypeStruct((MB, NH), jnp.int32),
    )(kb2, ql8, ca_last, sb_last)


NW = 32
BPW = NH * KSEL // NW


def _sc_gather(k2v, v2v, rids):
    mesh = plsc.VectorSubcoreMesh(core_axis_name="c", subcore_axis_name="s")

    @functools.partial(
        pl.kernel, mesh=mesh,
        out_type=[jax.ShapeDtypeStruct((NH * KSEL, HD), jnp.int32),
                  jax.ShapeDtypeStruct((NH * KSEL, HD), jnp.int32)],
        scratch_types=[pltpu.VMEM((BPW,), jnp.int32),
                       pltpu.VMEM((BPW, HD), jnp.int32),
                       pltpu.VMEM((BPW, HD), jnp.int32),
                       pltpu.SemaphoreType.DMA,
                       pltpu.SemaphoreType.DMA],
    )
    def gk(k_hbm, v_hbm, idx_hbm, ko_hbm, vo_hbm,
           idx_v, krows, vrows, sem1, sem2):
        wid = jax.lax.axis_index("s") * 2 + jax.lax.axis_index("c")
        base = wid * BPW
        pltpu.sync_copy(idx_hbm.at[pl.ds(base, BPW)], idx_v)
        c1 = pltpu.async_copy(k_hbm.at[idx_v], krows, sem1)
        c2 = pltpu.async_copy(v_hbm.at[idx_v], vrows, sem2)
        c1.wait()
        c2.wait()
        pltpu.sync_copy(krows, ko_hbm.at[pl.ds(base, BPW)])
        pltpu.sync_copy(vrows, vo_hbm.at[pl.ds(base, BPW)])

    return gk(k2v, v2v, rids)


def _attn_kernel(pos_ref, q_ref, ks_ref, vs_ref, o_ref):
    qt = pl.program_id(1)
    q = q_ref[:, :]
    tq = q.shape[0]
    att = jax.lax.dot_general(
        q, ks_ref[:, :], (((1,), (1,)), ((), ())),
        preferred_element_type=jnp.float32) * SCALE
    qpos = (qt * tq + jax.lax.broadcasted_iota(jnp.int32, (tq, 1), 0)
            ).astype(jnp.float32)
    allow = pos_ref[0, :, :] <= qpos
    att = jnp.where(allow, att, NEG)
    m = jnp.max(att, axis=1, keepdims=True)
    e = jnp.exp(att - m)
    denom = jnp.sum(e, axis=1, keepdims=True)
    y = jax.lax.dot_general(
        e.astype(jnp.bfloat16), vs_ref[:, :], (((1,), (0,)), ((), ())),
        preferred_element_type=jnp.float32)
    o_ref[:, :] = (y / denom).astype(o_ref.dtype)


def _attention(pos3, q2, k_sel, v_sel, tq):
    return pl.pallas_call(
        _attn_kernel,
        grid=(NH, S // tq),
        in_specs=[
            pl.BlockSpec((1, 1, KSEL), lambda h, i: (h, 0, 0)),
            pl.BlockSpec((tq, HD), lambda h, i: (i, h)),
            pl.BlockSpec((KSEL, HD), lambda h, i: (h, h % 2)),
            pl.BlockSpec((KSEL, HD), lambda h, i: (h, h % 2)),
        ],
        out_specs=pl.BlockSpec((tq, HD), lambda h, i: (i, h)),
        out_shape=jax.ShapeDtypeStruct((S, NH * HD), jnp.bfloat16),
    )(pos3, q2, k_sel, v_sel)


def _mm_kernel(a_ref, b_ref, o_ref):
    o_ref[:, :] = jax.lax.dot_general(
        a_ref[:, :], b_ref[:, :], (((1,), (0,)), ((), ())),
        preferred_element_type=jnp.float32)


def _out_proj(a, b, bm):
    m, k = a.shape
    k2, n = b.shape
    return pl.pallas_call(
        _mm_kernel,
        grid=(m // bm,),
        in_specs=[pl.BlockSpec((bm, k), lambda i: (i, 0)),
                  pl.BlockSpec((k, n), lambda i: (0, 0))],
        out_specs=pl.BlockSpec((bm, n), lambda i: (i, 0)),
        out_shape=jax.ShapeDtypeStruct((m, n), jnp.float32),
    )(a, b)


def kernel(x, freqs_cis, wqkv, wo, input_pos):
    x2 = x[0]
    w_all = wqkv.astype(jnp.bfloat16)

    c = freqs_cis[:, :, 0]
    s = freqs_cis[:, :, 1]
    ca2 = jnp.repeat(c, 2, axis=1)
    sb2 = jnp.stack([-s, s], axis=-1).reshape(S, HD)

    q_rot, k_rot, v2, kb2, ql8 = _qkv_rope(x2, w_all, ca2, sb2, 256)

    ca_last = ca2[S - 1].reshape(1, HD)
    sb_last = sb2[S - 1].reshape(1, HD)
    top = _route(kb2, ql8, ca_last, sb_last)

    fixed = jnp.concatenate([
        jnp.arange(SINK_B, dtype=jnp.int32),
        jnp.arange(WIN_START, CUR_BLOCK + 1, dtype=jnp.int32)])
    block_index = jnp.concatenate(
        [jnp.broadcast_to(fixed[None, :], (NH, SINK_B + WIN_B)),
         top.T], axis=1)
    pos_sel = (block_index[:, :, None] * BS
               + jnp.arange(BS, dtype=jnp.int32)[None, None, :]
               ).reshape(NH, KSEL)
    pos3 = pos_sel.reshape(NH, 1, KSEL).astype(jnp.float32)
    rids = (pos_sel * (NH // 2)
            + (jnp.arange(NH, dtype=jnp.int32) // 2)[:, None]).reshape(-1)

    k32 = jax.lax.bitcast_convert_type(
        k_rot.reshape(S * (NH // 2), HD, 2), jnp.int32)
    v32 = jax.lax.bitcast_convert_type(
        v2.reshape(S * (NH // 2), HD, 2), jnp.int32)
    ks32, vs32 = _sc_gather(k32, v32, rids)
    k_sel = jax.lax.bitcast_convert_type(
        ks32.reshape(NH * KSEL, HD, 1), jnp.bfloat16).reshape(NH * KSEL, 2 * HD)
    v_sel = jax.lax.bitcast_convert_type(
        vs32.reshape(NH * KSEL, HD, 1), jnp.bfloat16).reshape(NH * KSEL, 2 * HD)

    y2 = _attention(pos3, q_rot, k_sel, v_sel, 512)

    out = _out_proj(y2, wo.astype(jnp.bfloat16), 512)
    return out.reshape(1, S, DIM)


# speedup vs baseline: 16.4644x; 16.4644x over previous
import math

import jax
import jax.numpy as jnp
from jax.experimental import pallas as pl
from jax.experimental.pallas import tpu as pltpu

S = 4096
DIM = 2048
NH, HD = 16, 128
HH = HD // 2
BS = 8
TB = S // BS
SINK_B = 4
WIN_B = 4
CUR_BLOCK = TB - 1
WIN_START = CUR_BLOCK - WIN_B + 1
MB = 512 // BS
KL = SINK_B + WIN_B + MB
KSEL = KL * BS
SCALE = 1.0 / math.sqrt(HD)
NEG = -1e30
KOFF = NH * HD
VOFF = 2 * NH * HD


def _pair_swap(z):
    ev = jax.lax.broadcasted_iota(jnp.int32, z.shape, 1) % 2 == 0
    return jnp.where(ev, jnp.roll(z, -1, axis=1), jnp.roll(z, 1, axis=1))


def _qkvrope_kernel(x_ref, w_ref, ca_ref, sb_ref,
                    qr_ref, kr_ref, v_ref, kb_ref, ql_ref):
    i = pl.program_id(0)
    acc = jax.lax.dot_general(
        x_ref[:, :].astype(jnp.bfloat16), w_ref[:, :],
        (((1,), (0,)), ((), ())),
        preferred_element_type=jnp.float32)
    bm = acc.shape[0]
    ca = jnp.concatenate([ca_ref[:, :]] * NH, axis=1)
    sb = jnp.concatenate([sb_ref[:, :]] * NH, axis=1)
    q = acc[:, :KOFF]
    k = acc[:, KOFF:VOFF]
    qrot = q * ca + _pair_swap(q) * sb
    krot = k * ca + _pair_swap(k) * sb
    qr_ref[:, :] = qrot.astype(jnp.bfloat16)
    kr_ref[:, :] = krot.astype(jnp.bfloat16)
    kb_ref[:, :] = jnp.mean(krot.reshape(bm // BS, BS, KOFF), axis=1)
    v_ref[:, :] = acc[:, VOFF:].astype(jnp.bfloat16)

    @pl.when(i == pl.num_programs(0) - 1)
    def _():
        ql_ref[:, :] = acc[bm - 8:, :KOFF]


def _qkv_rope(x2, w_all, ca2, sb2, bm):
    return pl.pallas_call(
        _qkvrope_kernel,
        grid=(S // bm,),
        in_specs=[pl.BlockSpec((bm, DIM), lambda i: (i, 0)),
                  pl.BlockSpec((DIM, 3 * NH * HD), lambda i: (0, 0)),
                  pl.BlockSpec((bm, HD), lambda i: (i, 0)),
                  pl.BlockSpec((bm, HD), lambda i: (i, 0))],
        out_specs=[pl.BlockSpec((bm, NH * HD), lambda i: (i, 0)),
                   pl.BlockSpec((bm, NH * HD), lambda i: (i, 0)),
                   pl.BlockSpec((bm, NH * HD), lambda i: (i, 0)),
                   pl.BlockSpec((bm // BS, NH * HD), lambda i: (i, 0)),
                   pl.BlockSpec((8, NH * HD), lambda i: (0, 0))],
        out_shape=[jax.ShapeDtypeStruct((S, NH * HD), jnp.bfloat16),
                   jax.ShapeDtypeStruct((S, NH * HD), jnp.bfloat16),
                   jax.ShapeDtypeStruct((S, NH * HD), jnp.bfloat16),
                   jax.ShapeDtypeStruct((TB, NH * HD), jnp.float32),
                   jax.ShapeDtypeStruct((8, NH * HD), jnp.float32)],
    )(x2, w_all, ca2, sb2)


def _route_kernel(kb_ref, ql_ref, ca_ref, sb_ref, top_ref):
    ca = ca_ref[:, :]
    sb = sb_ref[:, :]
    qraw = ql_ref[7:8, :]
    cols = []
    for h in range(NH):
        qh = qraw[:, h * HD:(h + 1) * HD]
        qlh = qh * ca + _pair_swap(qh) * sb
        kb16 = kb_ref[:, h * HD:(h + 1) * HD].astype(jnp.bfloat16)
        ql16 = qlh.astype(jnp.bfloat16).astype(jnp.float32)
        cols.append(jnp.sum(kb16.astype(jnp.float32) * ql16,
                            axis=1, keepdims=True))
    scores = jnp.concatenate(cols, axis=1)
    rid = jax.lax.broadcasted_iota(jnp.int32, (TB, NH), 0)
    excl = (rid < SINK_B) | (rid >= WIN_START)
    scores = jnp.where(excl, NEG, scores)

    def body(j, sc):
        idx = jnp.argmax(sc, axis=0).astype(jnp.int32)
        top_ref[pl.ds(j, 1), :] = idx[None, :]
        hit = rid == idx[None, :]
        return jnp.where(hit, -jnp.inf, sc)

    jax.lax.fori_loop(0, MB, body, scores)


def _route(kb2, ql8, ca_last, sb_last):
    return pl.pallas_call(
        _route_kernel,
        grid=(1,),
        in_specs=[pl.BlockSpec((TB, NH * HD), lambda i: (0, 0)),
                  pl.BlockSpec((8, NH * HD), lambda i: (0, 0)),
                  pl.BlockSpec((1, HD), lambda i: (0, 0)),
                  pl.BlockSpec((1, HD), lambda i: (0, 0))],
        out_specs=pl.BlockSpec((MB, NH), lambda i: (0, 0)),
        out_shape=jax.ShapeDtypeStruct((MB, NH), jnp.int32),
    )(kb2, ql8, ca_last, sb_last)


def _attn_kernel(bi_ref, pos_ref, q_ref, k_ref, v_ref, o_ref, ks_ref, vs_ref):
    qt = pl.program_id(1)

    @pl.when(qt == 0)
    def _gather():
        def body(j, _):
            blk = bi_ref[0, 0, j]
            ks_ref[pl.ds(j * BS, BS), :] = k_ref[pl.ds(blk * BS, BS), :]
            vs_ref[pl.ds(j * BS, BS), :] = v_ref[pl.ds(blk * BS, BS), :]
            return 0
        jax.lax.fori_loop(0, KL, body, 0)

    q = q_ref[:, :]
    tq = q.shape[0]
    att = jax.lax.dot_general(
        q, ks_ref[:, :], (((1,), (1,)), ((), ())),
        preferred_element_type=jnp.float32) * SCALE
    qpos = (qt * tq + jax.lax.broadcasted_iota(jnp.int32, (tq, 1), 0)
            ).astype(jnp.float32)
    allow = pos_ref[0, :, :] <= qpos
    att = jnp.where(allow, att, NEG)
    m = jnp.max(att, axis=1, keepdims=True)
    e = jnp.exp(att - m)
    denom = jnp.sum(e, axis=1, keepdims=True)
    y = jax.lax.dot_general(
        e.astype(jnp.bfloat16), vs_ref[:, :], (((1,), (0,)), ((), ())),
        preferred_element_type=jnp.float32)
    o_ref[:, :] = (y / denom).astype(o_ref.dtype)


def _attention(block_index, pos3, q2, k2, v2, tq):
    return pl.pallas_call(
        _attn_kernel,
        grid=(NH, S // tq),
        in_specs=[
            pl.BlockSpec((1, 1, KL), lambda h, i: (h, 0, 0),
                         memory_space=pltpu.SMEM),
            pl.BlockSpec((1, 1, KSEL), lambda h, i: (h, 0, 0)),
            pl.BlockSpec((tq, HD), lambda h, i: (i, h)),
            pl.BlockSpec((S, HD), lambda h, i: (0, h)),
            pl.BlockSpec((S, HD), lambda h, i: (0, h)),
        ],
        out_specs=pl.BlockSpec((tq, HD), lambda h, i: (i, h)),
        out_shape=jax.ShapeDtypeStruct((S, NH * HD), jnp.bfloat16),
        scratch_shapes=[pltpu.VMEM((KSEL, HD), jnp.bfloat16),
                        pltpu.VMEM((KSEL, HD), jnp.bfloat16)],
    )(block_index.reshape(NH, 1, KL), pos3, q2, k2, v2)


def _mm_kernel(a_ref, b_ref, o_ref):
    o_ref[:, :] = jax.lax.dot_general(
        a_ref[:, :], b_ref[:, :], (((1,), (0,)), ((), ())),
        preferred_element_type=jnp.float32)


def _out_proj(a, b, bm):
    m, k = a.shape
    k2, n = b.shape
    return pl.pallas_call(
        _mm_kernel,
        grid=(m // bm,),
        in_specs=[pl.BlockSpec((bm, k), lambda i: (i, 0)),
                  pl.BlockSpec((k, n), lambda i: (0, 0))],
        out_specs=pl.BlockSpec((bm, n), lambda i: (i, 0)),
        out_shape=jax.ShapeDtypeStruct((m, n), jnp.float32),
    )(a, b)


def kernel(x, freqs_cis, wqkv, wo, input_pos):
    x2 = x[0]
    w_all = wqkv.astype(jnp.bfloat16)

    c = freqs_cis[:, :, 0]
    s = freqs_cis[:, :, 1]
    ca2 = jnp.repeat(c, 2, axis=1)
    sb2 = jnp.stack([-s, s], axis=-1).reshape(S, HD)

    q_rot, k_rot, v2, kb2, ql8 = _qkv_rope(x2, w_all, ca2, sb2, 256)

    ca_last = ca2[S - 1].reshape(1, HD)
    sb_last = sb2[S - 1].reshape(1, HD)
    top = _route(kb2, ql8, ca_last, sb_last)

    fixed = jnp.concatenate([
        jnp.arange(SINK_B, dtype=jnp.int32),
        jnp.arange(WIN_START, CUR_BLOCK + 1, dtype=jnp.int32)])
    block_index = jnp.concatenate(
        [jnp.broadcast_to(fixed[None, :], (NH, SINK_B + WIN_B)),
         top.T], axis=1)
    pos3 = (block_index[:, :, None] * BS
            + jnp.arange(BS, dtype=jnp.int32)[None, None, :]
            ).reshape(NH, 1, KSEL).astype(jnp.float32)

    y2 = _attention(block_index, pos3, q_rot, k_rot, v2, 512)

    out = _out_proj(y2, wo.astype(jnp.bfloat16), 512)
    return out.reshape(1, S, DIM)
